# baseline (speedup 1.0000x reference)
import functools

import jax
import jax.numpy as jnp
from jax import lax
from jax.experimental import pallas as pl
from jax.experimental.pallas import tpu as pltpu
from jax.experimental.pallas import tpu_sc as plsc

DIM = 256
KCODES = 8192
BETA = 0.25
TOK = 1024
BK = 256


def _argmin_body(z_ref, e_ref, ebf_ref, idx_ref, dsum_ref,
                 best_ref, bidx_ref, z2_ref, zbf_ref, cidx_ref):
    b = pl.program_id(0)
    k = pl.program_id(1)
    nk = pl.num_programs(1)

    e = e_ref[pl.ds(k * BK, BK), :]

    @pl.when(k == 0)
    def _():
        zt = z_ref[0]
        z2_ref[...] = jnp.sum(zt * zt, axis=0, keepdims=True)
        zbf_ref[...] = zt.astype(jnp.bfloat16)

    prod = lax.dot_general(
        ebf_ref[pl.ds(k * BK, BK), :], zbf_ref[...],
        (((1,), (0,)), ((), ())),
        preferred_element_type=jnp.float32,
    )
    e2 = jnp.sum(e * e, axis=1, keepdims=True)
    s = (z2_ref[...] + e2) + prod

    m = jnp.min(s, axis=0, keepdims=True)
    maskf = jnp.where(s == m, 1.0, 0.0)
    arow = lax.broadcasted_iota(jnp.int32, (2, BK), 1).astype(jnp.float32)
    sel0 = lax.broadcasted_iota(jnp.int32, (2, BK), 0) == 0
    wmat = jnp.where(sel0, arow, 1.0)
    cnts = lax.dot_general(wmat, maskf, (((1,), (0,)), ((), ())),
                           preferred_element_type=jnp.float32)
    cidx_ref[...] = cnts[0:1, :].astype(jnp.int32)

    @pl.when(jnp.max(cnts[1:2, :]) > 1.5)
    def _():
        rows = lax.broadcasted_iota(jnp.int32, (BK, TOK), 0)
        cidx_ref[...] = jnp.min(jnp.where(s == m, rows, BK), axis=0,
                                keepdims=True)

    cand = cidx_ref[...] + k * BK

    @pl.when(k == 0)
    def _():
        best_ref[...] = m
        bidx_ref[...] = cand

    @pl.when(k > 0)
    def _():
        prev = best_ref[...]
        take = m < prev
        best_ref[...] = jnp.where(take, m, prev)
        bidx_ref[...] = jnp.where(take, cand, bidx_ref[...])

    @pl.when(jnp.logical_and(b == 0, k == 0))
    def _():
        dsum_ref[0, 0] = 0.0

    @pl.when(k == nk - 1)
    def _():
        idx_ref[0] = bidx_ref[...]
        dsum_ref[0, 0] += jnp.sum(best_ref[...])


def _argmin_call(zr, emb):
    nb = zr.shape[0]
    nkb = KCODES // BK
    return pl.pallas_call(
        _argmin_body,
        grid=(nb, nkb),
        in_specs=[
            pl.BlockSpec((1, DIM, TOK), lambda b, k: (b, 0, 0)),
            pl.BlockSpec((KCODES, DIM), lambda b, k: (0, 0)),
            pl.BlockSpec((KCODES, DIM), lambda b, k: (0, 0)),
        ],
        out_specs=[
            pl.BlockSpec((1, 1, TOK), lambda b, k: (b, 0, 0)),
            pl.BlockSpec((1, 1), lambda b, k: (0, 0),
                         memory_space=pltpu.SMEM),
        ],
        out_shape=[
            jax.ShapeDtypeStruct((nb, 1, TOK), jnp.int32),
            jax.ShapeDtypeStruct((1, 1), jnp.float32),
        ],
        scratch_shapes=[
            pltpu.VMEM((1, TOK), jnp.float32),
            pltpu.VMEM((1, TOK), jnp.int32),
            pltpu.VMEM((1, TOK), jnp.float32),
            pltpu.VMEM((DIM, TOK), jnp.bfloat16),
            pltpu.VMEM((1, TOK), jnp.int32),
        ],
    )(zr, emb, (emb * -2.0).astype(jnp.bfloat16))


@functools.lru_cache(maxsize=None)
def _make_gather(V, D, B):
    info = plsc.get_sparse_core_info()
    NC, NS, L = info.num_cores, info.num_subcores, info.num_lanes
    NW = NC * NS
    assert D % L == 0 and B % (8 * NW) == 0
    b_per_w = B // NW
    mesh = plsc.VectorSubcoreMesh(core_axis_name="c", subcore_axis_name="s")

    @functools.partial(
        pl.kernel, mesh=mesh,
        out_type=jax.ShapeDtypeStruct((B, D), jnp.float32),
        scratch_types=[
            pltpu.VMEM((b_per_w,), jnp.int32),
            pltpu.VMEM((b_per_w, D), jnp.float32),
            pltpu.SemaphoreType.DMA,
        ],
    )
    def gather(table_hbm, idx_hbm, out_hbm, idx_v, rows_v, sem):
        wid = lax.axis_index("s") * NC + lax.axis_index("c")
        base = wid * b_per_w
        pltpu.sync_copy(idx_hbm.at[pl.ds(base, b_per_w)], idx_v)
        pltpu.async_copy(table_hbm.at[idx_v], rows_v, sem).wait()
        pltpu.sync_copy(rows_v, out_hbm.at[pl.ds(base, b_per_w)])

    return gather


def kernel(z, embedding):
    B, C, H, W = z.shape
    zr = z.reshape(B, C, H * W)
    idx3, dsum = _argmin_call(zr, embedding)
    idx = idx3.reshape(B * H * W)
    rows = _make_gather(KCODES, DIM, B * H * W)(embedding, idx)
    z_q = rows.reshape(B, H, W, C).transpose(0, 3, 1, 2)
    diff = dsum[0, 0] * (BETA / (B * H * W * C))
    return (z_q, diff, idx)


# speedup vs baseline: 1.1714x; 1.1714x over previous
import functools

import jax
import jax.numpy as jnp
from jax import lax
from jax.experimental import pallas as pl
from jax.experimental.pallas import tpu as pltpu
from jax.experimental.pallas import tpu_sc as plsc

DIM = 256
KCODES = 8192
BETA = 0.25
TOK = 1024
BK = 256


def _argmin_body(z_ref, e_ref, ebf_ref, idx_ref, dsum_ref,
                 best_ref, bidx_ref, z2_ref, zbf_ref):
    b = pl.program_id(0)
    k = pl.program_id(1)
    nk = pl.num_programs(1)

    e = e_ref[pl.ds(k * BK, BK), :]

    @pl.when(k == 0)
    def _():
        zt = z_ref[0]
        z2_ref[...] = jnp.sum(zt * zt, axis=0, keepdims=True)
        zbf_ref[...] = zt.astype(jnp.bfloat16)

    prod = lax.dot_general(
        ebf_ref[pl.ds(k * BK, BK), :], zbf_ref[...],
        (((1,), (0,)), ((), ())),
        preferred_element_type=jnp.float32,
    )
    e2 = jnp.sum(e * e, axis=1, keepdims=True)
    s = (z2_ref[...] + e2) + prod

    m = jnp.min(s, axis=0, keepdims=True)
    rows = lax.broadcasted_iota(jnp.int32, (BK, TOK), 0)
    cand = jnp.min(jnp.where(s == m, rows, BK), axis=0, keepdims=True) + k * BK

    @pl.when(k == 0)
    def _():
        best_ref[...] = m
        bidx_ref[...] = cand

    @pl.when(k > 0)
    def _():
        prev = best_ref[...]
        take = m < prev
        best_ref[...] = jnp.where(take, m, prev)
        bidx_ref[...] = jnp.where(take, cand, bidx_ref[...])

    @pl.when(jnp.logical_and(b == 0, k == 0))
    def _():
        dsum_ref[0, 0] = 0.0

    @pl.when(k == nk - 1)
    def _():
        idx_ref[0] = bidx_ref[...]
        dsum_ref[0, 0] += jnp.sum(best_ref[...])


def _argmin_call(zr, emb):
    nb = zr.shape[0]
    nkb = KCODES // BK
    return pl.pallas_call(
        _argmin_body,
        grid=(nb, nkb),
        in_specs=[
            pl.BlockSpec((1, DIM, TOK), lambda b, k: (b, 0, 0)),
            pl.BlockSpec((KCODES, DIM), lambda b, k: (0, 0)),
            pl.BlockSpec((KCODES, DIM), lambda b, k: (0, 0)),
        ],
        out_specs=[
            pl.BlockSpec((1, 1, TOK), lambda b, k: (b, 0, 0)),
            pl.BlockSpec((1, 1), lambda b, k: (0, 0),
                         memory_space=pltpu.SMEM),
        ],
        out_shape=[
            jax.ShapeDtypeStruct((nb, 1, TOK), jnp.int32),
            jax.ShapeDtypeStruct((1, 1), jnp.float32),
        ],
        scratch_shapes=[
            pltpu.VMEM((1, TOK), jnp.float32),
            pltpu.VMEM((1, TOK), jnp.int32),
            pltpu.VMEM((1, TOK), jnp.float32),
            pltpu.VMEM((DIM, TOK), jnp.bfloat16),
        ],
    )(zr, emb, (emb * -2.0).astype(jnp.bfloat16))


@functools.lru_cache(maxsize=None)
def _make_gather(V, D, B):
    info = plsc.get_sparse_core_info()
    NC, NS, L = info.num_cores, info.num_subcores, info.num_lanes
    NW = NC * NS
    assert D % L == 0 and B % (8 * NW) == 0
    b_per_w = B // NW
    mesh = plsc.VectorSubcoreMesh(core_axis_name="c", subcore_axis_name="s")

    @functools.partial(
        pl.kernel, mesh=mesh,
        out_type=jax.ShapeDtypeStruct((B, D), jnp.float32),
        scratch_types=[
            pltpu.VMEM((b_per_w,), jnp.int32),
            pltpu.VMEM((b_per_w, D), jnp.float32),
            pltpu.SemaphoreType.DMA,
        ],
    )
    def gather(table_hbm, idx_hbm, out_hbm, idx_v, rows_v, sem):
        wid = lax.axis_index("s") * NC + lax.axis_index("c")
        base = wid * b_per_w
        pltpu.sync_copy(idx_hbm.at[pl.ds(base, b_per_w)], idx_v)
        pltpu.async_copy(table_hbm.at[idx_v], rows_v, sem).wait()
        pltpu.sync_copy(rows_v, out_hbm.at[pl.ds(base, b_per_w)])

    return gather


def kernel(z, embedding):
    B, C, H, W = z.shape
    zr = z.reshape(B, C, H * W)
    idx3, dsum = _argmin_call(zr, embedding)
    idx = idx3.reshape(B * H * W)
    rows = _make_gather(KCODES, DIM, B * H * W)(embedding, idx)
    z_q = rows.reshape(B, H, W, C).transpose(0, 3, 1, 2)
    diff = dsum[0, 0] * (BETA / (B * H * W * C))
    return (z_q, diff, idx)
